# Initial kernel scaffold; baseline (speedup 1.0000x reference)
import jax
import jax.numpy as jnp
from jax.experimental import pallas as pl


def kernel(warped_xyz, warped_points, batch_info, batch_size, f2_xyz, f2_points, lidar_z, params):
    raise NotImplementedError("write your pallas kernel here")


# speedup vs baseline: 3.1951x; 3.1951x over previous
import functools

import jax
import jax.numpy as jnp
from jax.experimental import pallas as pl

_K = 16


def _knn_body(K, M, q_ref, st_ref, pen_ref, o_ref):
    q = q_ref[0]
    qq = jnp.sum(q * q, axis=1, keepdims=True)
    st = st_ref[0]
    s0, s1, s2 = st[0:1], st[1:2], st[2:3]
    ss = s0 * s0 + s1 * s1 + s2 * s2
    pen = pen_ref[0]
    cross = jax.lax.dot_general(
        q, st, dimension_numbers=(((1,), (0,)), ((), ())),
        preferred_element_type=jnp.float32)
    d = qq + ss - 2.0 * cross + pen
    iota = jax.lax.broadcasted_iota(jnp.int32, (1, M), 1)
    cols = []
    for _ in range(K):
        v = jnp.min(d, axis=1, keepdims=True)
        i = jnp.min(jnp.where(d == v, iota, M), axis=1, keepdims=True)
        cols.append(i)
        d = jnp.where(iota == i, jnp.inf, d)
    o_ref[0] = jnp.concatenate(cols, axis=1)


def _knn_topk(q, s, pen, K, TQ=128):
    B, NQ, _ = q.shape
    M = s.shape[1]
    st = jnp.transpose(s, (0, 2, 1))
    pen3 = pen.reshape(B, 1, M)
    body = functools.partial(_knn_body, K, M)
    return pl.pallas_call(
        body,
        grid=(B, NQ // TQ),
        in_specs=[
            pl.BlockSpec((1, TQ, 3), lambda b, j: (b, j, 0)),
            pl.BlockSpec((1, 3, M), lambda b, j: (b, 0, 0)),
            pl.BlockSpec((1, 1, M), lambda b, j: (b, 0, 0)),
        ],
        out_specs=pl.BlockSpec((1, TQ, K), lambda b, j: (b, j, 0)),
        out_shape=jax.ShapeDtypeStruct((B, NQ, K), jnp.int32),
    )(q, st, pen3)


def _nc2bnc(feats, batch_info, length):
    counts = jnp.bincount(batch_info, length=length)
    n = feats[0].shape[0]
    n_t = jnp.max(counts)
    offset = jnp.cumsum(counts) - counts
    ind = jnp.arange(n)
    new_count = jnp.full_like(counts, n)
    new_offset = jnp.cumsum(new_count) - new_count
    ind = ind + (new_offset - offset)[batch_info]
    out = []
    for feat in feats:
        c = feat.shape[-1]
        buf = jnp.zeros((length * n, c), dtype=jnp.float32).at[ind].set(feat)
        out.append(buf.reshape(length, n, c))
    return out, ind, n_t


def _mlp_layer(x, p, m, denom):
    W, gamma, beta = p
    y = jnp.einsum('bnkc,oc->bnko', x, W)
    mean = jnp.sum(y * m, axis=(0, 1, 2), keepdims=True) / denom
    var = jnp.sum((y - mean) * (y - mean) * m, axis=(0, 1, 2), keepdims=True) / denom
    y = (y - mean) / jnp.sqrt(var + 1e-5) * gamma + beta
    return jnp.where(y >= 0, y, 0.01 * y)


def _norm_ch(x):
    m = jnp.mean(x, axis=-1, keepdims=True)
    s = jnp.std(x, axis=-1, keepdims=True, ddof=1)
    return (x - m) / jnp.maximum(s, 1e-12)


def kernel(warped_xyz, warped_points, batch_info, batch_size, f2_xyz, f2_points, lidar_z, params):
    B = f2_xyz.shape[0]
    M = f2_xyz.shape[1]
    (wx, wp, lz), inv, n_t = _nc2bnc(
        [warped_xyz, warped_points, lidar_z], batch_info, B)
    valid_mask = (jnp.sum(wx * wx, axis=-1) >= 1e-10).astype(jnp.float32)

    idx_q = _knn_topk(wx, f2_xyz, jnp.zeros((B, M), jnp.float32), _K)
    bi = jnp.arange(B)[:, None, None]
    qi_xyz_grouped = f2_xyz[bi, idx_q]
    qi_points_grouped = f2_points[bi, idx_q]

    wx = wx * lz
    K = _K
    b, n, _ = wx.shape
    slot_mask = (jnp.arange(n) < n_t).astype(jnp.float32)[None, :, None, None]
    n_f = n_t.astype(jnp.float32)
    denom_q = b * n_f * K
    denom_p = b * n_f * _K
    pi_xyz_expanded = jnp.broadcast_to(wx[:, :, None, :], (b, n, K, 3))
    pi_points_expanded = jnp.broadcast_to(wp[:, :, None, :], (b, n, K, wp.shape[-1]))
    pi_xyz_diff_concat = jnp.concatenate([pi_xyz_expanded, qi_xyz_grouped], axis=3)
    pi_points_expanded = _norm_ch(pi_points_expanded)
    qi_points_grouped = _norm_ch(qi_points_grouped)
    pi_feat_diff = pi_points_expanded * qi_points_grouped
    pi_feat1_new = jnp.concatenate([pi_xyz_diff_concat, pi_feat_diff], axis=3)
    for p in params['mlp1']:
        pi_feat1_new = _mlp_layer(pi_feat1_new, p, slot_mask, denom_q)
    pi_xyz_encoding = _mlp_layer(pi_xyz_diff_concat, params['pi_enc'], slot_mask, denom_q)
    pi_concat = jnp.concatenate([pi_xyz_encoding, pi_feat1_new], axis=3)
    for p in params['mlp2']:
        pi_concat = _mlp_layer(pi_concat, p, slot_mask, denom_q)
    WQ = jax.nn.softmax(pi_concat, axis=2)
    pi_feat1_new = jnp.sum(WQ * pi_feat1_new, axis=2)

    pen = (1.0 - valid_mask) * 1e10
    idx_p = _knn_topk(wx, wx, pen, _K)
    pc_xyz_grouped = wx[bi, idx_p]
    pc_points_grouped = pi_feat1_new[bi, idx_p]

    pc_xyz_new = jnp.broadcast_to(wx[:, :, None, :], (b, n, _K, 3))
    pc_points_new = jnp.broadcast_to(wp[:, :, None, :], (b, n, _K, wp.shape[-1]))
    pc_xyz_diff = pc_xyz_grouped - pc_xyz_new
    pc_euc_diff = jnp.sqrt(jnp.sum(pc_xyz_diff * pc_xyz_diff, axis=3, keepdims=True) + 1e-20)
    pc_xyz_diff_concat = jnp.concatenate(
        [pc_xyz_new, pc_xyz_grouped, pc_xyz_diff, pc_euc_diff], axis=3)
    pc_xyz_encoding = _mlp_layer(pc_xyz_diff_concat, params['pc_enc'], slot_mask, denom_p)
    pc_concat = jnp.concatenate([pc_xyz_encoding, pc_points_new, pc_points_grouped], axis=-1)
    for p in params['mlp2b']:
        pc_concat = _mlp_layer(pc_concat, p, slot_mask, denom_p)
    WP = jax.nn.softmax(pc_concat, axis=2)
    pc_feat1_new = jnp.sum(WP * pc_points_grouped, axis=2)
    c = pc_feat1_new.shape[-1]
    return pc_feat1_new.reshape(b * n, c)[inv]


# speedup vs baseline: 12.0752x; 3.7793x over previous
import functools

import jax
import jax.numpy as jnp
from jax import lax
from jax.experimental import pallas as pl
from jax.experimental.pallas import tpu as pltpu
from jax.experimental.pallas import tpu_sc as plsc

_K = 16


def _sc_gather(table, idx, chunk=512):
    R = idx.shape[0]
    D = table.shape[1]
    info = plsc.get_sparse_core_info()
    nw = info.num_cores * info.num_subcores
    per_w = R // nw
    ch = min(chunk, per_w)
    mesh = plsc.VectorSubcoreMesh(core_axis_name="c", subcore_axis_name="s")

    @functools.partial(
        pl.kernel, mesh=mesh,
        out_type=jax.ShapeDtypeStruct((R, D), jnp.float32),
        scratch_types=[
            pltpu.VMEM((ch,), jnp.int32),
            pltpu.VMEM((ch, D), jnp.float32),
            pltpu.SemaphoreType.DMA,
        ],
    )
    def gk(table_hbm, idx_hbm, out_hbm, idx_v, rows_v, sem):
        wid = lax.axis_index("s") * info.num_cores + lax.axis_index("c")
        base = wid * per_w

        def body(i, carry):
            off = base + i * ch
            pltpu.sync_copy(idx_hbm.at[pl.ds(off, ch)], idx_v)
            pltpu.async_copy(table_hbm.at[idx_v], rows_v, sem).wait()
            pltpu.sync_copy(rows_v, out_hbm.at[pl.ds(off, ch)])
            return carry

        lax.fori_loop(0, per_w // ch, body, 0)

    return gk(table, idx)


def _knn_body(K, M, q_ref, st_ref, pen_ref, o_ref):
    q = q_ref[0]
    qq = jnp.sum(q * q, axis=1, keepdims=True)
    st = st_ref[0]
    s0, s1, s2 = st[0:1], st[1:2], st[2:3]
    ss = s0 * s0 + s1 * s1 + s2 * s2
    pen = pen_ref[0]
    cross = jax.lax.dot_general(
        q, st, dimension_numbers=(((1,), (0,)), ((), ())),
        preferred_element_type=jnp.float32)
    d = qq + ss - 2.0 * cross + pen
    iota = jax.lax.broadcasted_iota(jnp.int32, (1, M), 1)
    cols = []
    for _ in range(K):
        v = jnp.min(d, axis=1, keepdims=True)
        i = jnp.min(jnp.where(d == v, iota, M), axis=1, keepdims=True)
        cols.append(i)
        d = jnp.where(iota == i, jnp.inf, d)
    o_ref[0] = jnp.concatenate(cols, axis=1)


def _knn_topk(q, s, pen, K, TQ=128):
    B, NQ, _ = q.shape
    M = s.shape[1]
    st = jnp.transpose(s, (0, 2, 1))
    pen3 = pen.reshape(B, 1, M)
    body = functools.partial(_knn_body, K, M)
    return pl.pallas_call(
        body,
        grid=(B, NQ // TQ),
        in_specs=[
            pl.BlockSpec((1, TQ, 3), lambda b, j: (b, j, 0)),
            pl.BlockSpec((1, 3, M), lambda b, j: (b, 0, 0)),
            pl.BlockSpec((1, 1, M), lambda b, j: (b, 0, 0)),
        ],
        out_specs=pl.BlockSpec((1, TQ, K), lambda b, j: (b, j, 0)),
        out_shape=jax.ShapeDtypeStruct((B, NQ, K), jnp.int32),
    )(q, st, pen3)


def _nc2bnc(feats, batch_info, length):
    counts = jnp.bincount(batch_info, length=length)
    n = feats[0].shape[0]
    n_t = jnp.max(counts)
    offset = jnp.cumsum(counts) - counts
    ind = jnp.arange(n)
    new_count = jnp.full_like(counts, n)
    new_offset = jnp.cumsum(new_count) - new_count
    ind = ind + (new_offset - offset)[batch_info]
    out = []
    for feat in feats:
        c = feat.shape[-1]
        buf = jnp.zeros((length * n, c), dtype=jnp.float32).at[ind].set(feat)
        out.append(buf.reshape(length, n, c))
    return out, ind, n_t


def _mlp_layer(x, p, m, denom):
    W, gamma, beta = p
    y = jnp.einsum('bnkc,oc->bnko', x, W)
    mean = jnp.sum(y * m, axis=(0, 1, 2), keepdims=True) / denom
    var = jnp.sum((y - mean) * (y - mean) * m, axis=(0, 1, 2), keepdims=True) / denom
    y = (y - mean) / jnp.sqrt(var + 1e-5) * gamma + beta
    return jnp.where(y >= 0, y, 0.01 * y)


def _norm_ch(x):
    m = jnp.mean(x, axis=-1, keepdims=True)
    s = jnp.std(x, axis=-1, keepdims=True, ddof=1)
    return (x - m) / jnp.maximum(s, 1e-12)


def kernel(warped_xyz, warped_points, batch_info, batch_size, f2_xyz, f2_points, lidar_z, params):
    B = f2_xyz.shape[0]
    M = f2_xyz.shape[1]
    (wx, wp, lz), inv, n_t = _nc2bnc(
        [warped_xyz, warped_points, lidar_z], batch_info, B)
    valid_mask = (jnp.sum(wx * wx, axis=-1) >= 1e-10).astype(jnp.float32)

    idx_q = _knn_topk(wx, f2_xyz, jnp.zeros((B, M), jnp.float32), _K)
    bi = jnp.arange(B)[:, None, None]
    C = f2_points.shape[-1]
    n_all = wx.shape[1]
    tab1 = jnp.concatenate(
        [f2_points.reshape(B * M, C),
         jnp.pad(f2_xyz.reshape(B * M, 3), ((0, 0), (0, 125 - C)))], axis=1)
    flat_q = (idx_q + (bi * M)).reshape(-1)
    g1 = _sc_gather(tab1, flat_q)
    qi_points_grouped = g1[:, :C].reshape(B, n_all, _K, C)
    qi_xyz_grouped = g1[:, C:C + 3].reshape(B, n_all, _K, 3)

    wx = wx * lz
    K = _K
    b, n, _ = wx.shape
    slot_mask = (jnp.arange(n) < n_t).astype(jnp.float32)[None, :, None, None]
    n_f = n_t.astype(jnp.float32)
    denom_q = b * n_f * K
    denom_p = b * n_f * _K
    pi_xyz_expanded = jnp.broadcast_to(wx[:, :, None, :], (b, n, K, 3))
    pi_points_expanded = jnp.broadcast_to(wp[:, :, None, :], (b, n, K, wp.shape[-1]))
    pi_xyz_diff_concat = jnp.concatenate([pi_xyz_expanded, qi_xyz_grouped], axis=3)
    pi_points_expanded = _norm_ch(pi_points_expanded)
    qi_points_grouped = _norm_ch(qi_points_grouped)
    pi_feat_diff = pi_points_expanded * qi_points_grouped
    pi_feat1_new = jnp.concatenate([pi_xyz_diff_concat, pi_feat_diff], axis=3)
    for p in params['mlp1']:
        pi_feat1_new = _mlp_layer(pi_feat1_new, p, slot_mask, denom_q)
    pi_xyz_encoding = _mlp_layer(pi_xyz_diff_concat, params['pi_enc'], slot_mask, denom_q)
    pi_concat = jnp.concatenate([pi_xyz_encoding, pi_feat1_new], axis=3)
    for p in params['mlp2']:
        pi_concat = _mlp_layer(pi_concat, p, slot_mask, denom_q)
    WQ = jax.nn.softmax(pi_concat, axis=2)
    pi_feat1_new = jnp.sum(WQ * pi_feat1_new, axis=2)

    pen = (1.0 - valid_mask) * 1e10
    idx_p = _knn_topk(wx, wx, pen, _K)
    c2 = pi_feat1_new.shape[-1]
    tab2 = jnp.concatenate(
        [pi_feat1_new.reshape(b * n, c2),
         jnp.pad(wx.reshape(b * n, 3), ((0, 0), (0, 125 - c2)))], axis=1)
    flat_p = (idx_p + (bi * n)).reshape(-1)
    g2 = _sc_gather(tab2, flat_p)
    pc_points_grouped = g2[:, :c2].reshape(b, n, _K, c2)
    pc_xyz_grouped = g2[:, c2:c2 + 3].reshape(b, n, _K, 3)

    pc_xyz_new = jnp.broadcast_to(wx[:, :, None, :], (b, n, _K, 3))
    pc_points_new = jnp.broadcast_to(wp[:, :, None, :], (b, n, _K, wp.shape[-1]))
    pc_xyz_diff = pc_xyz_grouped - pc_xyz_new
    pc_euc_diff = jnp.sqrt(jnp.sum(pc_xyz_diff * pc_xyz_diff, axis=3, keepdims=True) + 1e-20)
    pc_xyz_diff_concat = jnp.concatenate(
        [pc_xyz_new, pc_xyz_grouped, pc_xyz_diff, pc_euc_diff], axis=3)
    pc_xyz_encoding = _mlp_layer(pc_xyz_diff_concat, params['pc_enc'], slot_mask, denom_p)
    pc_concat = jnp.concatenate([pc_xyz_encoding, pc_points_new, pc_points_grouped], axis=-1)
    for p in params['mlp2b']:
        pc_concat = _mlp_layer(pc_concat, p, slot_mask, denom_p)
    WP = jax.nn.softmax(pc_concat, axis=2)
    pc_feat1_new = jnp.sum(WP * pc_points_grouped, axis=2)
    c = pc_feat1_new.shape[-1]
    flat_out = jnp.pad(pc_feat1_new.reshape(b * n, c), ((0, 0), (0, 128 - c)))
    return _sc_gather(flat_out, inv.astype(jnp.int32), chunk=256)[:, :c]
